# baseline (speedup 1.0000x reference)
import functools

import jax
import jax.numpy as jnp
from jax import lax
from jax.experimental import pallas as pl
from jax.experimental.pallas import tpu as pltpu
from jax.experimental.pallas import tpu_sc as plsc

B = 16384
D = 64
SCN = 8192
TCN = B - SCN
NC = 2
NS = 16
NW = NC * NS
BPW = SCN // NW
LANES = 16
SLABW = 128
VPG = 16
NHALF = 2
NVH = BPW // VPG // NHALF
RING = 8
PPS = 8


def _extract_column(slabs, rows, slot, i, lane):
    lane_v = jnp.full((LANES,), lane, dtype=jnp.int32)
    slot_v = jnp.full((LANES,), slot, dtype=jnp.int32)
    iota = lax.iota(jnp.int32, LANES)
    for jj in range(D // LANES):
        k_v = iota + (jj * LANES)
        vals = plsc.load_gather(slabs, [slot_v, k_v, lane_v])
        rows[i, pl.ds(jj * LANES, LANES)] = vals


def _gather_body(uet, bet, xu, xb, u_out, b_out,
                 idxu, idxb, slabs, urows, brows, sem_a, sem_b):
    wid = lax.axis_index("s") * NC + lax.axis_index("c")
    base = wid * BPW
    pltpu.sync_copy(xu.at[pl.ds(base, BPW)], idxu)
    pltpu.sync_copy(xb.at[pl.ds(base, BPW)], idxb)

    def fire(tbl, r, slot, sem):
        c = lax.div(r, SLABW)
        return pltpu.async_copy(
            tbl.at[:, pl.ds(pl.multiple_of(c * SLABW, SLABW), SLABW)],
            slabs.at[slot], sem)

    def drain(slot, sem):
        pltpu.make_async_copy(uet.at[:, pl.ds(0, SLABW)],
                              slabs.at[slot], sem).wait()

    sems = [sem_a, sem_b]

    for half in range(NHALF):
        def body(vv, carry, half=half):
            off = (half * NVH + vv) * VPG
            ivu = idxu[pl.ds(pl.multiple_of(off, VPG), VPG)]
            ivb = idxb[pl.ds(pl.multiple_of(off, VPG), VPG)]
            rsu = [ivu[b] for b in range(VPG)]
            rsb = [ivb[b] for b in range(VPG)]

            def fire4(g, sem):
                h = (g % 2) * 4
                fire(uet, rsu[2 * g], h, sem)
                fire(uet, rsu[2 * g + 1], h + 1, sem)
                fire(bet, rsb[2 * g], h + 2, sem)
                fire(bet, rsb[2 * g + 1], h + 3, sem)

            fire4(0, sems[0])
            for g in range(VPG // 2):
                cur = sems[g % 2]
                ch = (g % 2) * 4
                if g + 1 < VPG // 2:
                    fire4(g + 1, sems[1 - g % 2])
                for s in range(4):
                    drain(ch + s, cur)
                iloc = vv * VPG + 2 * g
                _extract_column(slabs, urows, ch, iloc,
                                lax.rem(rsu[2 * g], SLABW))
                _extract_column(slabs, urows, ch + 1, iloc + 1,
                                lax.rem(rsu[2 * g + 1], SLABW))
                _extract_column(slabs, brows, ch + 2, iloc,
                                lax.rem(rsb[2 * g], SLABW))
                _extract_column(slabs, brows, ch + 3, iloc + 1,
                                lax.rem(rsb[2 * g + 1], SLABW))
            return carry

        lax.fori_loop(0, NVH, body, 0)
        hb = base + half * (BPW // NHALF)
        pltpu.sync_copy(urows, u_out.at[pl.ds(hb, BPW // NHALF)])
        pltpu.sync_copy(brows, b_out.at[pl.ds(hb, BPW // NHALF)])


def _sc_gather(uet, bet, xu, xb):
    mesh = plsc.VectorSubcoreMesh(
        core_axis_name="c", subcore_axis_name="s",
        num_cores=NC, num_subcores=NS)
    f = pl.kernel(
        _gather_body,
        out_type=(jax.ShapeDtypeStruct((SCN, D), jnp.float32),
                  jax.ShapeDtypeStruct((SCN, D), jnp.float32)),
        mesh=mesh,
        compiler_params=pltpu.CompilerParams(needs_layout_passes=False),
        scratch_types=[
            pltpu.VMEM((BPW,), jnp.int32),
            pltpu.VMEM((BPW,), jnp.int32),
            pltpu.VMEM((RING, D, SLABW), jnp.float32),
            pltpu.VMEM((BPW // NHALF, D), jnp.float32),
            pltpu.VMEM((BPW // NHALF, D), jnp.float32),
            pltpu.SemaphoreType.DMA,
            pltpu.SemaphoreType.DMA,
        ],
    )
    return f(uet, bet, xu, xb)


def _tc_gather_body(cols_ref, lanes_ref, *refs):
    ublks = refs[:PPS]
    bblks = refs[PPS:2 * PPS]
    uout, bout = refs[2 * PPS], refs[2 * PPS + 1]
    i = pl.program_id(0)
    iota = lax.broadcasted_iota(jnp.int32, (1, SLABW), 1)
    for j in range(PPS):
        lu = lanes_ref[0, i * PPS + j]
        lb = lanes_ref[1, i * PPS + j]
        ohu = (iota == lu).astype(jnp.float32)
        ohb = (iota == lb).astype(jnp.float32)
        dn = (((1,), (1,)), ((), ()))
        ub = ublks[j][:]
        ub = jnp.where(jnp.isfinite(ub), ub, 0.0)
        bb = bblks[j][:]
        bb = jnp.where(jnp.isfinite(bb), bb, 0.0)
        uout[pl.ds(j, 1), :] = lax.dot_general(
            ohu, ub, dn, preferred_element_type=jnp.float32)
        bout[pl.ds(j, 1), :] = lax.dot_general(
            ohb, bb, dn, preferred_element_type=jnp.float32)


def _tc_gather(uet, bet, xu_tc, xb_tc):
    cols = jnp.stack([xu_tc // SLABW, xb_tc // SLABW])
    lanes = jnp.stack([xu_tc % SLABW, xb_tc % SLABW])
    tbl_spec = [
        pl.BlockSpec((D, SLABW),
                     (lambda i, cols_ref, lanes_ref, t=t, j=j:
                      (0, cols_ref[t, i * PPS + j])))
        for t in range(2) for j in range(PPS)
    ]
    out_spec = pl.BlockSpec((PPS, D), lambda i, cols_ref, lanes_ref: (i, 0))
    grid_spec = pltpu.PrefetchScalarGridSpec(
        num_scalar_prefetch=2,
        grid=(TCN // PPS,),
        in_specs=[tbl_spec[j] for j in range(PPS)] +
                 [tbl_spec[PPS + j] for j in range(PPS)],
        out_specs=[out_spec, out_spec],
    )
    return pl.pallas_call(
        _tc_gather_body,
        grid_spec=grid_spec,
        out_shape=(jax.ShapeDtypeStruct((TCN, D), jnp.float32),
                   jax.ShapeDtypeStruct((TCN, D), jnp.float32)),
    )(cols, lanes, *([uet] * PPS), *([bet] * PPS))


def _mlp_body(usc, btc_u, bsc, btc_b, fcw_ref, fcb_ref, hlw_ref, hlb_ref,
              out_ref):
    def half(u, b):
        h = jnp.dot(u, fcw_ref[0:D, :], preferred_element_type=jnp.float32)
        h = h + jnp.dot(b, fcw_ref[D:2 * D, :],
                        preferred_element_type=jnp.float32)
        h = h + fcb_ref[:]
        o = (jnp.dot(h, hlw_ref[:], preferred_element_type=jnp.float32)
             + hlb_ref[:])
        return 4.0 * jax.nn.sigmoid(o) + 1.0

    out_ref[0:SCN, :] = half(usc[:], bsc[:])
    out_ref[SCN:B, :] = half(btc_u[:], btc_b[:])


def _tc_mlp(u_sc, u_tc, b_sc, b_tc, fc_w, fc_b, hl_w, hl_b):
    return pl.pallas_call(
        _mlp_body,
        out_shape=jax.ShapeDtypeStruct((B, 5), jnp.float32),
    )(u_sc, u_tc, b_sc, b_tc, fc_w, fc_b.reshape(1, -1),
      hl_w, hl_b.reshape(1, -1))


def kernel(x, user_emb, book_emb, fc_w, fc_b, hl_w, hl_b):
    xu = x[:, 0]
    xb = x[:, 1]
    uet = user_emb.T
    bet = book_emb.T
    u_sc, b_sc = _sc_gather(uet, bet, xu[:SCN], xb[:SCN])
    u_tc, b_tc = _tc_gather(uet, bet, xu[SCN:], xb[SCN:])
    return _tc_mlp(u_sc, u_tc, b_sc, b_tc, fc_w, fc_b, hl_w, hl_b)


# speedup vs baseline: 1.9635x; 1.9635x over previous
import functools

import jax
import jax.numpy as jnp
from jax import lax
from jax.experimental import pallas as pl
from jax.experimental.pallas import tpu as pltpu
from jax.experimental.pallas import tpu_sc as plsc

B = 16384
D = 64
SCN = 11264
TCN = B - SCN
NC = 2
NS = 16
NW = NC * NS
BPW = SCN // NW
LANES = 16
SLABW = 128
VPG = 16
NHALF = 2
NVH = BPW // VPG // NHALF
RING = 8
PPS = 16


def _extract_column(slabs, rows, slot, i, lane):
    lane_v = jnp.full((LANES,), lane, dtype=jnp.int32)
    slot_v = jnp.full((LANES,), slot, dtype=jnp.int32)
    iota = lax.iota(jnp.int32, LANES)
    for jj in range(D // LANES):
        k_v = iota + (jj * LANES)
        vals = plsc.load_gather(slabs, [slot_v, k_v, lane_v])
        rows[i, pl.ds(jj * LANES, LANES)] = vals


def _gather_body(uet, bet, xu, xb, u_out, b_out,
                 idxu, idxb, slabs, urows, brows, sem_a, sem_b):
    wid = lax.axis_index("s") * NC + lax.axis_index("c")
    base = wid * BPW
    pltpu.sync_copy(xu.at[pl.ds(base, BPW)], idxu)
    pltpu.sync_copy(xb.at[pl.ds(base, BPW)], idxb)

    def fire(tbl, r, slot, sem):
        c = lax.div(r, SLABW)
        return pltpu.async_copy(
            tbl.at[:, pl.ds(pl.multiple_of(c * SLABW, SLABW), SLABW)],
            slabs.at[slot], sem)

    def drain(slot, sem):
        pltpu.make_async_copy(uet.at[:, pl.ds(0, SLABW)],
                              slabs.at[slot], sem).wait()

    sems = [sem_a, sem_b]

    for half in range(NHALF):
        def body(vv, carry, half=half):
            off = (half * NVH + vv) * VPG
            ivu = idxu[pl.ds(pl.multiple_of(off, VPG), VPG)]
            ivb = idxb[pl.ds(pl.multiple_of(off, VPG), VPG)]
            rsu = [ivu[b] for b in range(VPG)]
            rsb = [ivb[b] for b in range(VPG)]

            def fire4(g, sem):
                h = (g % 2) * 4
                fire(uet, rsu[2 * g], h, sem)
                fire(uet, rsu[2 * g + 1], h + 1, sem)
                fire(bet, rsb[2 * g], h + 2, sem)
                fire(bet, rsb[2 * g + 1], h + 3, sem)

            fire4(0, sems[0])
            for g in range(VPG // 2):
                cur = sems[g % 2]
                ch = (g % 2) * 4
                if g + 1 < VPG // 2:
                    fire4(g + 1, sems[1 - g % 2])
                for s in range(4):
                    drain(ch + s, cur)
                iloc = vv * VPG + 2 * g
                _extract_column(slabs, urows, ch, iloc,
                                lax.rem(rsu[2 * g], SLABW))
                _extract_column(slabs, urows, ch + 1, iloc + 1,
                                lax.rem(rsu[2 * g + 1], SLABW))
                _extract_column(slabs, brows, ch + 2, iloc,
                                lax.rem(rsb[2 * g], SLABW))
                _extract_column(slabs, brows, ch + 3, iloc + 1,
                                lax.rem(rsb[2 * g + 1], SLABW))
            return carry

        lax.fori_loop(0, NVH, body, 0)
        hb = base + half * (BPW // NHALF)
        pltpu.sync_copy(urows, u_out.at[pl.ds(hb, BPW // NHALF)])
        pltpu.sync_copy(brows, b_out.at[pl.ds(hb, BPW // NHALF)])


def _sc_gather(uet, bet, xu, xb):
    mesh = plsc.VectorSubcoreMesh(
        core_axis_name="c", subcore_axis_name="s",
        num_cores=NC, num_subcores=NS)
    f = pl.kernel(
        _gather_body,
        out_type=(jax.ShapeDtypeStruct((SCN, D), jnp.float32),
                  jax.ShapeDtypeStruct((SCN, D), jnp.float32)),
        mesh=mesh,
        compiler_params=pltpu.CompilerParams(needs_layout_passes=False),
        scratch_types=[
            pltpu.VMEM((BPW,), jnp.int32),
            pltpu.VMEM((BPW,), jnp.int32),
            pltpu.VMEM((RING, D, SLABW), jnp.float32),
            pltpu.VMEM((BPW // NHALF, D), jnp.float32),
            pltpu.VMEM((BPW // NHALF, D), jnp.float32),
            pltpu.SemaphoreType.DMA,
            pltpu.SemaphoreType.DMA,
        ],
    )
    return f(uet, bet, xu, xb)


def _tc_gather_body(cols_ref, lanes_ref, *refs):
    ublks = refs[:PPS]
    bblks = refs[PPS:2 * PPS]
    uout, bout = refs[2 * PPS], refs[2 * PPS + 1]
    i = pl.program_id(0)
    iota = lax.broadcasted_iota(jnp.int32, (1, SLABW), 1)
    for j in range(PPS):
        lu = lanes_ref[0, i * PPS + j]
        lb = lanes_ref[1, i * PPS + j]
        ohu = (iota == lu).astype(jnp.float32)
        ohb = (iota == lb).astype(jnp.float32)
        dn = (((1,), (1,)), ((), ()))
        ub = ublks[j][:]
        ub = jnp.where(jnp.isfinite(ub), ub, 0.0)
        bb = bblks[j][:]
        bb = jnp.where(jnp.isfinite(bb), bb, 0.0)
        uout[pl.ds(j, 1), :] = lax.dot_general(
            ohu, ub, dn, preferred_element_type=jnp.float32)
        bout[pl.ds(j, 1), :] = lax.dot_general(
            ohb, bb, dn, preferred_element_type=jnp.float32)


def _tc_gather(uet, bet, xu_tc, xb_tc):
    cols = jnp.stack([xu_tc // SLABW, xb_tc // SLABW])
    lanes = jnp.stack([xu_tc % SLABW, xb_tc % SLABW])
    tbl_spec = [
        pl.BlockSpec((D, SLABW),
                     (lambda i, cols_ref, lanes_ref, t=t, j=j:
                      (0, cols_ref[t, i * PPS + j])))
        for t in range(2) for j in range(PPS)
    ]
    out_spec = pl.BlockSpec((PPS, D), lambda i, cols_ref, lanes_ref: (i, 0))
    grid_spec = pltpu.PrefetchScalarGridSpec(
        num_scalar_prefetch=2,
        grid=(TCN // PPS,),
        in_specs=[tbl_spec[j] for j in range(PPS)] +
                 [tbl_spec[PPS + j] for j in range(PPS)],
        out_specs=[out_spec, out_spec],
    )
    return pl.pallas_call(
        _tc_gather_body,
        grid_spec=grid_spec,
        out_shape=(jax.ShapeDtypeStruct((TCN, D), jnp.float32),
                   jax.ShapeDtypeStruct((TCN, D), jnp.float32)),
    )(cols, lanes, *([uet] * PPS), *([bet] * PPS))


def _mlp_body(usc, btc_u, bsc, btc_b, fcw_ref, fcb_ref, hlw_ref, hlb_ref,
              out_ref):
    def half(u, b):
        h = jnp.dot(u, fcw_ref[0:D, :], preferred_element_type=jnp.float32)
        h = h + jnp.dot(b, fcw_ref[D:2 * D, :],
                        preferred_element_type=jnp.float32)
        h = h + fcb_ref[:]
        o = (jnp.dot(h, hlw_ref[:], preferred_element_type=jnp.float32)
             + hlb_ref[:])
        return 4.0 * jax.nn.sigmoid(o) + 1.0

    out_ref[0:SCN, :] = half(usc[:], bsc[:])
    out_ref[SCN:B, :] = half(btc_u[:], btc_b[:])


def _tc_mlp(u_sc, u_tc, b_sc, b_tc, fc_w, fc_b, hl_w, hl_b):
    return pl.pallas_call(
        _mlp_body,
        out_shape=jax.ShapeDtypeStruct((B, 5), jnp.float32),
    )(u_sc, u_tc, b_sc, b_tc, fc_w, fc_b.reshape(1, -1),
      hl_w, hl_b.reshape(1, -1))


def kernel(x, user_emb, book_emb, fc_w, fc_b, hl_w, hl_b):
    xu = x[:, 0]
    xb = x[:, 1]
    uet = user_emb.T
    bet = book_emb.T
    u_sc, b_sc = _sc_gather(uet, bet, xu[:SCN], xb[:SCN])
    u_tc, b_tc = _tc_gather(uet, bet, xu[SCN:], xb[SCN:])
    return _tc_mlp(u_sc, u_tc, b_sc, b_tc, fc_w, fc_b, hl_w, hl_b)


# speedup vs baseline: 2.0534x; 1.0458x over previous
import functools

import jax
import jax.numpy as jnp
from jax import lax
from jax.experimental import pallas as pl
from jax.experimental.pallas import tpu as pltpu
from jax.experimental.pallas import tpu_sc as plsc

B = 16384
D = 64
SCN = 12288
TCN = B - SCN
NC = 2
NS = 16
NW = NC * NS
BPW = SCN // NW
LANES = 16
SLABW = 128
VPG = 16
NHALF = 2
NVH = BPW // VPG // NHALF
RING = 8
PPS = 16


def _extract_column(slabs, rows, slot, i, lane):
    lane_v = jnp.full((LANES,), lane, dtype=jnp.int32)
    slot_v = jnp.full((LANES,), slot, dtype=jnp.int32)
    iota = lax.iota(jnp.int32, LANES)
    for jj in range(D // LANES):
        k_v = iota + (jj * LANES)
        vals = plsc.load_gather(slabs, [slot_v, k_v, lane_v])
        rows[i, pl.ds(jj * LANES, LANES)] = vals


def _gather_body(uet, bet, xu, xb, u_out, b_out,
                 idxu, idxb, slabs, urows, brows, sem_a, sem_b):
    wid = lax.axis_index("s") * NC + lax.axis_index("c")
    base = wid * BPW
    pltpu.sync_copy(xu.at[pl.ds(base, BPW)], idxu)
    pltpu.sync_copy(xb.at[pl.ds(base, BPW)], idxb)

    def fire(tbl, r, slot, sem):
        c = lax.div(r, SLABW)
        return pltpu.async_copy(
            tbl.at[:, pl.ds(pl.multiple_of(c * SLABW, SLABW), SLABW)],
            slabs.at[slot], sem)

    def drain(slot, sem):
        pltpu.make_async_copy(uet.at[:, pl.ds(0, SLABW)],
                              slabs.at[slot], sem).wait()

    sems = [sem_a, sem_b]

    for half in range(NHALF):
        def body(vv, carry, half=half):
            off = (half * NVH + vv) * VPG
            ivu = idxu[pl.ds(pl.multiple_of(off, VPG), VPG)]
            ivb = idxb[pl.ds(pl.multiple_of(off, VPG), VPG)]
            rsu = [ivu[b] for b in range(VPG)]
            rsb = [ivb[b] for b in range(VPG)]

            def fire4(g, sem):
                h = (g % 2) * 4
                fire(uet, rsu[2 * g], h, sem)
                fire(uet, rsu[2 * g + 1], h + 1, sem)
                fire(bet, rsb[2 * g], h + 2, sem)
                fire(bet, rsb[2 * g + 1], h + 3, sem)

            fire4(0, sems[0])
            for g in range(VPG // 2):
                cur = sems[g % 2]
                ch = (g % 2) * 4
                if g + 1 < VPG // 2:
                    fire4(g + 1, sems[1 - g % 2])
                for s in range(4):
                    drain(ch + s, cur)
                iloc = vv * VPG + 2 * g
                _extract_column(slabs, urows, ch, iloc,
                                lax.rem(rsu[2 * g], SLABW))
                _extract_column(slabs, urows, ch + 1, iloc + 1,
                                lax.rem(rsu[2 * g + 1], SLABW))
                _extract_column(slabs, brows, ch + 2, iloc,
                                lax.rem(rsb[2 * g], SLABW))
                _extract_column(slabs, brows, ch + 3, iloc + 1,
                                lax.rem(rsb[2 * g + 1], SLABW))
            return carry

        lax.fori_loop(0, NVH, body, 0)
        hb = base + half * (BPW // NHALF)
        pltpu.sync_copy(urows, u_out.at[pl.ds(hb, BPW // NHALF)])
        pltpu.sync_copy(brows, b_out.at[pl.ds(hb, BPW // NHALF)])


def _sc_gather(uet, bet, xu, xb):
    mesh = plsc.VectorSubcoreMesh(
        core_axis_name="c", subcore_axis_name="s",
        num_cores=NC, num_subcores=NS)
    f = pl.kernel(
        _gather_body,
        out_type=(jax.ShapeDtypeStruct((SCN, D), jnp.float32),
                  jax.ShapeDtypeStruct((SCN, D), jnp.float32)),
        mesh=mesh,
        compiler_params=pltpu.CompilerParams(needs_layout_passes=False),
        scratch_types=[
            pltpu.VMEM((BPW,), jnp.int32),
            pltpu.VMEM((BPW,), jnp.int32),
            pltpu.VMEM((RING, D, SLABW), jnp.float32),
            pltpu.VMEM((BPW // NHALF, D), jnp.float32),
            pltpu.VMEM((BPW // NHALF, D), jnp.float32),
            pltpu.SemaphoreType.DMA,
            pltpu.SemaphoreType.DMA,
        ],
    )
    return f(uet, bet, xu, xb)


def _tc_gather_body(cols_ref, lanes_ref, *refs):
    ublks = refs[:PPS]
    bblks = refs[PPS:2 * PPS]
    uout, bout = refs[2 * PPS], refs[2 * PPS + 1]
    i = pl.program_id(0)
    iota = lax.broadcasted_iota(jnp.int32, (1, SLABW), 1)
    ones = jnp.ones((1, SLABW), dtype=jnp.float32)
    dn = (((1,), (1,)), ((), ()))
    for j in range(PPS):
        lu = lanes_ref[0, i * PPS + j]
        lb = lanes_ref[1, i * PPS + j]
        ub = jnp.where(iota == lu, ublks[j][:], 0.0)
        bb = jnp.where(iota == lb, bblks[j][:], 0.0)
        uout[pl.ds(j, 1), :] = lax.dot_general(
            ones, ub, dn, preferred_element_type=jnp.float32)
        bout[pl.ds(j, 1), :] = lax.dot_general(
            ones, bb, dn, preferred_element_type=jnp.float32)


def _tc_gather(uet, bet, xu_tc, xb_tc):
    cols = jnp.stack([xu_tc // SLABW, xb_tc // SLABW])
    lanes = jnp.stack([xu_tc % SLABW, xb_tc % SLABW])
    tbl_spec = [
        pl.BlockSpec((D, SLABW),
                     (lambda i, cols_ref, lanes_ref, t=t, j=j:
                      (0, cols_ref[t, i * PPS + j])))
        for t in range(2) for j in range(PPS)
    ]
    out_spec = pl.BlockSpec((PPS, D), lambda i, cols_ref, lanes_ref: (i, 0))
    grid_spec = pltpu.PrefetchScalarGridSpec(
        num_scalar_prefetch=2,
        grid=(TCN // PPS,),
        in_specs=[tbl_spec[j] for j in range(PPS)] +
                 [tbl_spec[PPS + j] for j in range(PPS)],
        out_specs=[out_spec, out_spec],
    )
    return pl.pallas_call(
        _tc_gather_body,
        grid_spec=grid_spec,
        out_shape=(jax.ShapeDtypeStruct((TCN, D), jnp.float32),
                   jax.ShapeDtypeStruct((TCN, D), jnp.float32)),
    )(cols, lanes, *([uet] * PPS), *([bet] * PPS))


def _mlp_body(usc, btc_u, bsc, btc_b, fcw_ref, fcb_ref, hlw_ref, hlb_ref,
              out_ref):
    def half(u, b):
        h = jnp.dot(u, fcw_ref[0:D, :], preferred_element_type=jnp.float32)
        h = h + jnp.dot(b, fcw_ref[D:2 * D, :],
                        preferred_element_type=jnp.float32)
        h = h + fcb_ref[:]
        o = (jnp.dot(h, hlw_ref[:], preferred_element_type=jnp.float32)
             + hlb_ref[:])
        return 4.0 * jax.nn.sigmoid(o) + 1.0

    out_ref[0:SCN, :] = half(usc[:], bsc[:])
    out_ref[SCN:B, :] = half(btc_u[:], btc_b[:])


def _tc_mlp(u_sc, u_tc, b_sc, b_tc, fc_w, fc_b, hl_w, hl_b):
    return pl.pallas_call(
        _mlp_body,
        out_shape=jax.ShapeDtypeStruct((B, 5), jnp.float32),
    )(u_sc, u_tc, b_sc, b_tc, fc_w, fc_b.reshape(1, -1),
      hl_w, hl_b.reshape(1, -1))


def kernel(x, user_emb, book_emb, fc_w, fc_b, hl_w, hl_b):
    xu = x[:, 0]
    xb = x[:, 1]
    uet = user_emb.T
    bet = book_emb.T
    u_sc, b_sc = _sc_gather(uet, bet, xu[:SCN], xb[:SCN])
    u_tc, b_tc = _tc_gather(uet, bet, xu[SCN:], xb[SCN:])
    return _tc_mlp(u_sc, u_tc, b_sc, b_tc, fc_w, fc_b, hl_w, hl_b)
